# Initial kernel scaffold; baseline (speedup 1.0000x reference)
import jax
import jax.numpy as jnp
from jax.experimental import pallas as pl


def kernel(pos, edge_index, shift):
    raise NotImplementedError("write your pallas kernel here")


# speedup vs baseline: 1.7114x; 1.7114x over previous
import functools
import math

import jax
import jax.numpy as jnp
from jax import lax
from jax.experimental import pallas as pl
from jax.experimental.pallas import tpu as pltpu
from jax.experimental.pallas import tpu_sc as plsc

_N_NODES = 50_000
_N_EDGES = 3_200_000
_NW = 32
_EPW = _N_EDGES // _NW
_BLK = 1024
_NFULL = _EPW // _BLK
_TAIL = _EPW - _NFULL * _BLK
_CHUNK = 128

_S3 = math.sqrt(3.0)
_S5 = math.sqrt(5.0)


def _rsqrt(n):
    i = plsc.bitcast(n, jnp.int32)
    i = jnp.int32(0x5F3759DF) - (i >> 1)
    y = plsc.bitcast(i, jnp.float32)
    for _ in range(2):
        y = y * (jnp.float32(1.5) - jnp.float32(0.5) * n * y * y)
    return y


def _compute_block(xtab, ytab, idx_s, idx_d, z_s, z_d, shf,
                   obuf_v, obuf_l, obuf_sh, nb):
    iota = lax.iota(jnp.int32, 16)

    def grp(g, carry):
        ln = g * 16 + iota
        l3 = ln * 3
        l9 = ln * 9
        ns = idx_s[pl.ds(g * 16, 16)]
        nd = idx_d[pl.ds(g * 16, 16)]
        sx = plsc.load_gather(xtab, [ns])
        sy = plsc.load_gather(ytab, [ns])
        sz = z_s[pl.ds(g * 16, 16)]
        dxr = plsc.load_gather(xtab, [nd])
        dyr = plsc.load_gather(ytab, [nd])
        dzr = z_d[pl.ds(g * 16, 16)]
        hx = plsc.load_gather(shf, [l3])
        hy = plsc.load_gather(shf, [l3 + 1])
        hz = plsc.load_gather(shf, [l3 + 2])
        vx = dxr - sx + hx
        vy = dyr - sy + hy
        vz = dzr - sz + hz
        n = vx * vx + vy * vy + vz * vz
        r = _rsqrt(n)
        r = jnp.where(n > 0.0, r, jnp.float32(0.0))
        length = n * r
        ux = vx * r
        uy = vy * r
        uz = vz * r
        plsc.store_scatter(obuf_v, [l3], vx)
        plsc.store_scatter(obuf_v, [l3 + 1], vy)
        plsc.store_scatter(obuf_v, [l3 + 2], vz)
        obuf_l[pl.ds(g * 16, 16)] = length
        uxx = ux * ux
        uyy = uy * uy
        uzz = uz * uz
        s3 = jnp.float32(_S3)
        s5 = jnp.float32(_S5)
        plsc.store_scatter(obuf_sh, [l9], jnp.full((16,), 1.0, jnp.float32))
        plsc.store_scatter(obuf_sh, [l9 + 1], s3 * ux)
        plsc.store_scatter(obuf_sh, [l9 + 2], s3 * uy)
        plsc.store_scatter(obuf_sh, [l9 + 3], s3 * uz)
        plsc.store_scatter(obuf_sh, [l9 + 4], s5 * s3 * ux * uz)
        plsc.store_scatter(obuf_sh, [l9 + 5], s5 * s3 * ux * uy)
        plsc.store_scatter(obuf_sh, [l9 + 6],
                           s5 * (uyy - jnp.float32(0.5) * (uxx + uzz)))
        plsc.store_scatter(obuf_sh, [l9 + 7], s5 * s3 * uy * uz)
        plsc.store_scatter(obuf_sh, [l9 + 8],
                           s5 * jnp.float32(0.5 * _S3) * (uzz - uxx))
        return carry

    lax.fori_loop(0, nb // 16, grp, 0)


def _do_block(srcs, dsts, shiftf, ov, ol, osh, z_sp,
              xtab, ytab, idx_s, idx_d, z_s, z_d, shf,
              obuf_v, obuf_l, obuf_sh, sem, base, nb):
    pltpu.sync_copy(srcs.at[pl.ds(base, nb)], idx_s.at[pl.ds(0, nb)])
    pltpu.sync_copy(dsts.at[pl.ds(base, nb)], idx_d.at[pl.ds(0, nb)])
    pltpu.sync_copy(shiftf.at[pl.ds(base * 3, nb * 3)],
                    shf.at[pl.ds(0, nb * 3)])
    cps = []
    nfull, rem = divmod(nb, _CHUNK)
    for j in range(nfull):
        o = j * _CHUNK
        cps.append(pltpu.async_copy(
            z_sp.at[idx_s.at[pl.ds(o, _CHUNK)]], z_s.at[pl.ds(o, _CHUNK)],
            sem))
        cps.append(pltpu.async_copy(
            z_sp.at[idx_d.at[pl.ds(o, _CHUNK)]], z_d.at[pl.ds(o, _CHUNK)],
            sem))
    if rem:
        o = nfull * _CHUNK
        cps.append(pltpu.async_copy(
            z_sp.at[idx_s.at[pl.ds(o, rem)]], z_s.at[pl.ds(o, rem)], sem))
        cps.append(pltpu.async_copy(
            z_sp.at[idx_d.at[pl.ds(o, rem)]], z_d.at[pl.ds(o, rem)], sem))
    for c in cps:
        c.wait()
    _compute_block(xtab, ytab, idx_s, idx_d, z_s, z_d, shf,
                   obuf_v, obuf_l, obuf_sh, nb)
    pltpu.sync_copy(obuf_v.at[pl.ds(0, nb * 3)], ov.at[pl.ds(base * 3, nb * 3)])
    pltpu.sync_copy(obuf_l.at[pl.ds(0, nb)], ol.at[pl.ds(base, nb)])
    pltpu.sync_copy(obuf_sh.at[pl.ds(0, nb * 9)],
                    osh.at[pl.ds(base * 9, nb * 9)])


def _sc_body(xs, ys, zs, srcs, dsts, shiftf, ov, ol, osh,
             xtab, ytab, z_sp, idx_s, idx_d, z_s, z_d, shf,
             obuf_v, obuf_l, obuf_sh, sem):
    sid = lax.axis_index("s")
    cid = lax.axis_index("c")
    wid = sid * 2 + cid

    @pl.when(sid == 0)
    def _():
        pltpu.sync_copy(zs, z_sp)

    pltpu.sync_copy(xs, xtab)
    pltpu.sync_copy(ys, ytab)
    plsc.subcore_barrier()

    start = wid * _EPW

    def blk(b, carry):
        _do_block(srcs, dsts, shiftf, ov, ol, osh, z_sp,
                  xtab, ytab, idx_s, idx_d, z_s, z_d, shf,
                  obuf_v, obuf_l, obuf_sh, sem, start + b * _BLK, _BLK)
        return carry

    lax.fori_loop(0, _NFULL, blk, 0)
    _do_block(srcs, dsts, shiftf, ov, ol, osh, z_sp,
              xtab, ytab, idx_s, idx_d, z_s, z_d, shf,
              obuf_v, obuf_l, obuf_sh, sem, start + _NFULL * _BLK, _TAIL)


@jax.jit
def _run(xs, ys, zs, srcs, dsts, shiftf):
    mesh = plsc.VectorSubcoreMesh(core_axis_name="c", subcore_axis_name="s")
    f = pl.kernel(
        _sc_body,
        out_type=[
            jax.ShapeDtypeStruct((_N_EDGES * 3,), jnp.float32),
            jax.ShapeDtypeStruct((_N_EDGES,), jnp.float32),
            jax.ShapeDtypeStruct((_N_EDGES * 9,), jnp.float32),
        ],
        mesh=mesh,
        compiler_params=pltpu.CompilerParams(needs_layout_passes=False),
        scratch_types=[
            pltpu.VMEM((_N_NODES,), jnp.float32),
            pltpu.VMEM((_N_NODES,), jnp.float32),
            pltpu.VMEM_SHARED((_N_NODES,), jnp.float32),
            pltpu.VMEM((_BLK,), jnp.int32),
            pltpu.VMEM((_BLK,), jnp.int32),
            pltpu.VMEM((_BLK,), jnp.float32),
            pltpu.VMEM((_BLK,), jnp.float32),
            pltpu.VMEM((_BLK * 3,), jnp.float32),
            pltpu.VMEM((_BLK * 3,), jnp.float32),
            pltpu.VMEM((_BLK,), jnp.float32),
            pltpu.VMEM((_BLK * 9,), jnp.float32),
            pltpu.SemaphoreType.DMA,
        ],
    )
    return f(xs, ys, zs, srcs, dsts, shiftf)


def kernel(pos, edge_index, shift):
    xs = pos[:, 0]
    ys = pos[:, 1]
    zs = pos[:, 2]
    srcs = edge_index[0].astype(jnp.int32)
    dsts = edge_index[1].astype(jnp.int32)
    shiftf = shift.reshape(-1)
    ev, el, esh = _run(xs, ys, zs, srcs, dsts, shiftf)
    return (ev.reshape(_N_EDGES, 3), el, esh.reshape(_N_EDGES, 9))


# speedup vs baseline: 19.7135x; 11.5188x over previous
import functools
import math

import jax
import jax.numpy as jnp
from jax import lax
from jax.experimental import pallas as pl
from jax.experimental.pallas import tpu as pltpu
from jax.experimental.pallas import tpu_sc as plsc

_N_NODES = 50_000
_N_EDGES = 3_200_000
_NW = 32
_EPW = _N_EDGES // _NW
_BLK = 1024
_NFULL = _EPW // _BLK
_TAIL = _EPW - _NFULL * _BLK
_CHUNK = 128

_S3 = math.sqrt(3.0)
_S5 = math.sqrt(5.0)


def _rsqrt(n):
    i = plsc.bitcast(n, jnp.int32)
    i = jnp.int32(0x5F3759DF) - (i >> 1)
    y = plsc.bitcast(i, jnp.float32)
    for _ in range(2):
        y = y * (jnp.float32(1.5) - jnp.float32(0.5) * n * y * y)
    return y


def _compute_block(xtab, ytab, idx_s, idx_d, z_s, z_d, hbx, hby, hbz,
                   bvx, bvy, bvz, bl, bsh, nb):
    def grp(g, carry):
        o = pl.ds(g * 16, 16)
        ns = idx_s[o]
        nd = idx_d[o]
        sx = plsc.load_gather(xtab, [ns])
        sy = plsc.load_gather(ytab, [ns])
        sz = z_s[o]
        dxr = plsc.load_gather(xtab, [nd])
        dyr = plsc.load_gather(ytab, [nd])
        dzr = z_d[o]
        vx = dxr - sx + hbx[o]
        vy = dyr - sy + hby[o]
        vz = dzr - sz + hbz[o]
        n = vx * vx + vy * vy + vz * vz
        r = _rsqrt(n)
        r = jnp.where(n > 0.0, r, jnp.float32(0.0))
        ux = vx * r
        uy = vy * r
        uz = vz * r
        bvx[o] = vx
        bvy[o] = vy
        bvz[o] = vz
        bl[o] = n * r
        s3 = jnp.float32(_S3)
        s5 = jnp.float32(_S5)
        bsh[0][o] = s3 * ux
        bsh[1][o] = s3 * uy
        bsh[2][o] = s3 * uz
        bsh[3][o] = s5 * s3 * ux * uz
        bsh[4][o] = s5 * s3 * ux * uy
        bsh[5][o] = s5 * (uy * uy - jnp.float32(0.5) * (ux * ux + uz * uz))
        bsh[6][o] = s5 * s3 * uy * uz
        bsh[7][o] = s5 * jnp.float32(0.5 * _S3) * (uz * uz - ux * ux)
        return carry

    lax.fori_loop(0, nb // 16, grp, 0)


def _do_block(ins, outs, z_sp, xtab, ytab, scratch, sem, base, nb):
    srcs, dsts, shx, shy, shz = ins
    idx_s, idx_d, z_s, z_d, hbx, hby, hbz, bvx, bvy, bvz, bl, *bsh = scratch
    hbm_in = [srcs, dsts, shx, shy, shz]
    loc_in = [idx_s, idx_d, hbx, hby, hbz]
    pltpu.sync_copy(srcs.at[pl.ds(base, nb)], idx_s.at[pl.ds(0, nb)])
    pltpu.sync_copy(dsts.at[pl.ds(base, nb)], idx_d.at[pl.ds(0, nb)])
    cps = []
    nfull, rem = divmod(nb, _CHUNK)
    for j in range(nfull + (1 if rem else 0)):
        o = j * _CHUNK
        c = rem if (rem and j == nfull) else _CHUNK
        cps.append(pltpu.async_copy(
            z_sp.at[idx_s.at[pl.ds(o, c)]], z_s.at[pl.ds(o, c)], sem))
        cps.append(pltpu.async_copy(
            z_sp.at[idx_d.at[pl.ds(o, c)]], z_d.at[pl.ds(o, c)], sem))
    for hbm_ref, loc_ref in zip(hbm_in[2:], loc_in[2:]):
        pltpu.sync_copy(hbm_ref.at[pl.ds(base, nb)], loc_ref.at[pl.ds(0, nb)])
    for c in cps:
        c.wait()
    _compute_block(xtab, ytab, idx_s, idx_d, z_s, z_d, hbx, hby, hbz,
                   bvx, bvy, bvz, bl, bsh, nb)
    for hbm_ref, loc_ref in zip(outs, [bvx, bvy, bvz, bl, *bsh]):
        pltpu.sync_copy(loc_ref.at[pl.ds(0, nb)], hbm_ref.at[pl.ds(base, nb)])


def _sc_body(xs, ys, zs, srcs, dsts, shx, shy, shz, *out_and_scratch):
    outs = out_and_scratch[:12]
    xtab, ytab, z_sp, *scratch, sem = out_and_scratch[12:]
    sid = lax.axis_index("s")
    cid = lax.axis_index("c")
    wid = sid * 2 + cid

    @pl.when(sid == 0)
    def _():
        pltpu.sync_copy(zs, z_sp)

    pltpu.sync_copy(xs, xtab)
    pltpu.sync_copy(ys, ytab)
    plsc.subcore_barrier()

    start = wid * _EPW
    ins = (srcs, dsts, shx, shy, shz)

    def blk(b, carry):
        _do_block(ins, outs, z_sp, xtab, ytab, scratch, sem,
                  start + b * _BLK, _BLK)
        return carry

    lax.fori_loop(0, _NFULL, blk, 0)
    _do_block(ins, outs, z_sp, xtab, ytab, scratch, sem,
              start + _NFULL * _BLK, _TAIL)


@jax.jit
def _run(xs, ys, zs, srcs, dsts, shx, shy, shz):
    mesh = plsc.VectorSubcoreMesh(core_axis_name="c", subcore_axis_name="s")
    plane = jax.ShapeDtypeStruct((_N_EDGES,), jnp.float32)
    f = pl.kernel(
        _sc_body,
        out_type=[plane] * 12,
        mesh=mesh,
        compiler_params=pltpu.CompilerParams(needs_layout_passes=False),
        scratch_types=[
            pltpu.VMEM((_N_NODES,), jnp.float32),
            pltpu.VMEM((_N_NODES,), jnp.float32),
            pltpu.VMEM_SHARED((_N_NODES,), jnp.float32),
            pltpu.VMEM((_BLK,), jnp.int32),
            pltpu.VMEM((_BLK,), jnp.int32),
        ] + [pltpu.VMEM((_BLK,), jnp.float32)] * 17 + [
            pltpu.SemaphoreType.DMA,
        ],
    )
    return f(xs, ys, zs, srcs, dsts, shx, shy, shz)


def kernel(pos, edge_index, shift):
    xs = pos[:, 0]
    ys = pos[:, 1]
    zs = pos[:, 2]
    srcs = edge_index[0].astype(jnp.int32)
    dsts = edge_index[1].astype(jnp.int32)
    shx = shift[:, 0]
    shy = shift[:, 1]
    shz = shift[:, 2]
    vx, vy, vz, el, *sh = _run(xs, ys, zs, srcs, dsts, shx, shy, shz)
    ev = jnp.stack([vx, vy, vz], axis=1)
    esh = jnp.stack([jnp.ones_like(el), *sh], axis=1)
    return (ev, el, esh)
